# baseline (speedup 1.0000x reference)
import jax
import jax.numpy as jnp
from jax.experimental import pallas as pl

K = 8192
D = 256
_ROWS_PER_BLOCK = 2048


def _normalize_body(x_ref, o_ref):
    x = x_ref[...]
    sq = x * x
    ones = jnp.ones((D, 1), jnp.float32)
    s = jax.lax.dot_general(sq, ones, (((1,), (0,)), ((), ())),
                            preferred_element_type=jnp.float32)
    inv = 1.0 / jnp.maximum(jnp.sqrt(s), 1e-12)
    o_ref[...] = x * inv


def kernel(prototypes):
    return pl.pallas_call(
        _normalize_body,
        grid=(K // _ROWS_PER_BLOCK,),
        in_specs=[pl.BlockSpec((_ROWS_PER_BLOCK, D), lambda i: (i, 0))],
        out_specs=pl.BlockSpec((_ROWS_PER_BLOCK, D), lambda i: (i, 0)),
        out_shape=jax.ShapeDtypeStruct((K, D), jnp.float32),
    )(prototypes)
